# Initial kernel scaffold; baseline (speedup 1.0000x reference)
import jax
import jax.numpy as jnp
from jax.experimental import pallas as pl


def kernel(inputs, gate_w, expert_w):
    raise NotImplementedError("write your pallas kernel here")


# speedup vs baseline: 1.3711x; 1.3711x over previous
import math

import jax
import jax.numpy as jnp
from jax.experimental import pallas as pl
from jax.experimental.pallas import tpu as pltpu

S, D, E, N = 2048, 1024, 8, 2048
CAP = math.floor(1.25 * S / E)
CAP += CAP % 2
CAP = max(CAP, 4)
N_TILE = 512
NT = N // N_TILE


def _cumsum_tokens(m_bf16, lower_tri_bf16):
    return jax.lax.dot_general(
        lower_tri_bf16, m_bf16, (((1,), (0,)), ((), ())),
        preferred_element_type=jnp.float32).astype(jnp.int32)


def _moe_kernel(x_ref, gw_ref, w_ref, o_ref, dest1_s, dest2_s, disp_s):
    e = pl.program_id(0)
    nt = pl.program_id(1)

    @pl.when((e == 0) & (nt == 0))
    def _routing():
        x = x_ref[...]
        gw = gw_ref[...]
        logits = jax.lax.dot_general(
            x, gw, (((1,), (1,)), ((), ())),
            preferred_element_type=jnp.float32)
        lane = jax.lax.broadcasted_iota(jnp.int32, (S, E), 1)
        m1v = jnp.max(logits, axis=1, keepdims=True)
        idx1 = jnp.min(jnp.where(logits == m1v, lane, E), axis=1,
                       keepdims=True)
        mask1 = lane == idx1
        l2 = jnp.where(mask1, -jnp.inf, logits)
        m2v = jnp.max(l2, axis=1, keepdims=True)
        idx2 = jnp.min(jnp.where(l2 == m2v, lane, E), axis=1, keepdims=True)
        mask2 = lane == idx2
        row = jax.lax.broadcasted_iota(jnp.int32, (S, 1), 0)
        col = jax.lax.broadcasted_iota(jnp.int32, (1, S), 1)
        ltri = (col <= row).astype(jnp.bfloat16)
        m1b = mask1.astype(jnp.bfloat16)
        m2b = mask2.astype(jnp.bfloat16)
        rank1 = _cumsum_tokens(m1b, ltri) - 1
        n1 = jnp.sum(mask1.astype(jnp.int32), axis=0, keepdims=True)
        rank2 = _cumsum_tokens(m2b, ltri) - 1 + n1
        ok1 = mask1 & (rank1 < CAP)
        ok2 = mask2 & (rank2 < CAP)
        d1 = jnp.sum(jnp.where(ok1, lane * CAP + rank1, 0), axis=1,
                     keepdims=True)
        v1 = jnp.sum(ok1.astype(jnp.int32), axis=1, keepdims=True)
        d2 = jnp.sum(jnp.where(ok2, lane * CAP + rank2, 0), axis=1,
                     keepdims=True)
        v2 = jnp.sum(ok2.astype(jnp.int32), axis=1, keepdims=True)
        dest1_s[...] = jnp.where(v1 > 0, d1, -1)
        dest2_s[...] = jnp.where(v2 > 0, d2, -1)

    @pl.when(nt == 0)
    def _dispatch():
        cc = jax.lax.broadcasted_iota(jnp.int32, (S, CAP), 1) + e * CAP
        d1 = dest1_s[...]
        d2 = dest2_s[...]
        p = ((d1 == cc) | (d2 == cc)).astype(jnp.float32)
        disp_s[...] = jax.lax.dot_general(
            p, x_ref[...], (((0,), (0,)), ((), ())),
            preferred_element_type=jnp.float32)

    o_ref[...] = jax.lax.dot_general(
        disp_s[...], w_ref[0], (((1,), (0,)), ((), ())),
        preferred_element_type=jnp.float32)[None]


def kernel(inputs, gate_w, expert_w):
    return pl.pallas_call(
        _moe_kernel,
        grid=(E, NT),
        in_specs=[
            pl.BlockSpec((S, D), lambda e, n: (0, 0)),
            pl.BlockSpec((E, D), lambda e, n: (0, 0)),
            pl.BlockSpec((1, D, N_TILE), lambda e, n: (e, 0, n)),
        ],
        out_specs=pl.BlockSpec((1, CAP, N_TILE), lambda e, n: (e, 0, n)),
        out_shape=jax.ShapeDtypeStruct((E, CAP, N), jnp.float32),
        scratch_shapes=[
            pltpu.VMEM((S, 1), jnp.int32),
            pltpu.VMEM((S, 1), jnp.int32),
            pltpu.VMEM((CAP, D), jnp.float32),
        ],
    )(inputs, gate_w, expert_w)
